# baseline (speedup 1.0000x reference)
import functools

import jax
import jax.numpy as jnp
from jax import lax
from jax.experimental import pallas as pl
from jax.experimental.pallas import tpu as pltpu
from jax.experimental.pallas import tpu_sc as plsc

_ROWS = 16384
_COLS = 256
_NOUT = 8
_W = 32
_NC = 2
_NS = 16
_NW = _NC * _NS
_RPW = _ROWS // _NW
_RC = 128


def _sc_split_body(x_hbm, *rest):
    outs = rest[:_NOUT]
    buf = rest[_NOUT]
    sem = rest[_NOUT + 1]
    wid = lax.axis_index("s") * _NC + lax.axis_index("c")
    base = wid * _RPW
    for h in range(_RPW // _RC):
        rb = base + h * _RC
        pltpu.make_async_copy(
            x_hbm.at[pl.ds(rb, _RC)], buf, sem).start()
        pltpu.make_async_copy(
            x_hbm.at[pl.ds(rb, _RC)], buf, sem).wait()
        for i in range(_NOUT):
            pltpu.sync_copy(
                buf.at[:, pl.ds(i * _W, _W)], outs[i].at[pl.ds(rb, _RC)])


@jax.jit
def kernel(x):
    mesh = plsc.VectorSubcoreMesh(core_axis_name="c", subcore_axis_name="s")
    out_type = tuple(
        jax.ShapeDtypeStruct((_ROWS, _W), jnp.float32) for _ in range(_NOUT))
    scratch = [
        pltpu.VMEM((_RC, _COLS), jnp.float32),
        pltpu.SemaphoreType.DMA,
    ]
    f = pl.kernel(
        _sc_split_body,
        out_type=out_type,
        mesh=mesh,
        scratch_types=scratch,
        compiler_params=pltpu.CompilerParams(use_tc_tiling_on_sc=False),
    )
    return f(x)
